# baseline (speedup 1.0000x reference)
import functools

import jax
import jax.numpy as jnp
from jax.experimental import pallas as pl
import jax.experimental.pallas.tpu as pltpu

_K = 32
_NEG_DIAG = -1e9
_NEG_TAKEN = -3e9


def _knn_kernel(a0_ref, a1_ref, b_ref, s0o_ref, i0o_ref, s1o_ref, i1o_ref,
                s0_ref, s1_ref, col_ref, *, rblk, seq, k):
    t = pl.program_id(1)
    b_id = pl.program_id(0)

    @pl.when((b_id == 0) & (t == 0))
    def _():
        col_ref[...] = jax.lax.broadcasted_iota(
            jnp.int32, (rblk, seq), 1).astype(jnp.float32)

    b = b_ref[0]
    col = jax.lax.broadcasted_iota(jnp.int32, (rblk, seq), 1)
    row0 = (2 * t) * rblk + jax.lax.broadcasted_iota(
        jnp.int32, (rblk, seq), 0)

    s0 = jax.lax.dot_general(a0_ref[0], b, (((1,), (1,)), ((), ())),
                             preferred_element_type=jnp.float32)
    s0_ref[...] = jnp.where(col == row0, _NEG_DIAG, s0)
    s1 = jax.lax.dot_general(a1_ref[0], b, (((1,), (1,)), ((), ())),
                             preferred_element_type=jnp.float32)
    s1_ref[...] = jnp.where(col == row0 + rblk, _NEG_DIAG, s1)

    kcol = jax.lax.broadcasted_iota(jnp.int32, (rblk, k), 1)

    def one_round(s_ref, colf, kk, vals, idxs):
        s = s_ref[...]
        m = jnp.max(s, axis=1)
        cand = jnp.where(s >= m[:, None], colf, 3.0e9)
        posf = jnp.min(cand, axis=1)
        s_ref[...] = jnp.where(cand == posf[:, None], _NEG_TAKEN, s)
        pos = posf.astype(jnp.int32)
        sel = kcol == kk
        return (jnp.where(sel, m[:, None], vals),
                jnp.where(sel, pos[:, None], idxs))

    def body(kk, carry):
        v0, i0, v1, i1 = carry
        colf = col_ref[...]
        v0, i0 = one_round(s0_ref, colf, kk, v0, i0)
        v1, i1 = one_round(s1_ref, colf, kk, v1, i1)
        return v0, i0, v1, i1

    z = jnp.full((rblk, k), 0.0, jnp.float32)
    zi = jnp.full((rblk, k), 0, jnp.int32)
    v0, i0, v1, i1 = jax.lax.fori_loop(0, k, body, (z, zi, z, zi))
    s0o_ref[0, 0] = v0
    i0o_ref[0, 0] = i0
    s1o_ref[0, 0] = v1
    i1o_ref[0, 0] = i1


@jax.jit
def kernel(embeddings):
    batch, seq, d = embeddings.shape
    k = min(_K, seq - 1)
    rblk = min(256, max(1, seq // 2))
    nblk = seq // rblk
    assert nblk % 2 == 0 and rblk * nblk == seq
    npair = nblk // 2

    emb_n = embeddings / (
        jnp.linalg.norm(embeddings, axis=-1, keepdims=True) + 1e-08)

    kfn = functools.partial(_knn_kernel, rblk=rblk, seq=seq, k=k)
    outs = pl.pallas_call(
        kfn,
        grid=(batch, npair),
        in_specs=[
            pl.BlockSpec((1, rblk, d), lambda b, t: (b, 2 * t, 0)),
            pl.BlockSpec((1, rblk, d), lambda b, t: (b, 2 * t + 1, 0)),
            pl.BlockSpec((1, seq, d), lambda b, t: (b, 0, 0)),
        ],
        out_specs=[
            pl.BlockSpec((1, 1, rblk, k), lambda b, t: (b, t, 0, 0)),
            pl.BlockSpec((1, 1, rblk, k), lambda b, t: (b, t, 0, 0)),
            pl.BlockSpec((1, 1, rblk, k), lambda b, t: (b, t, 0, 0)),
            pl.BlockSpec((1, 1, rblk, k), lambda b, t: (b, t, 0, 0)),
        ],
        out_shape=[
            jax.ShapeDtypeStruct((batch, npair, rblk, k), jnp.float32),
            jax.ShapeDtypeStruct((batch, npair, rblk, k), jnp.int32),
            jax.ShapeDtypeStruct((batch, npair, rblk, k), jnp.float32),
            jax.ShapeDtypeStruct((batch, npair, rblk, k), jnp.int32),
        ],
        scratch_shapes=[pltpu.VMEM((rblk, seq), jnp.float32),
                        pltpu.VMEM((rblk, seq), jnp.float32),
                        pltpu.VMEM((rblk, seq), jnp.float32)],
    )(emb_n, emb_n, emb_n)
    s0o, i0o, s1o, i1o = outs

    scores = jnp.stack([s0o, s1o], axis=2).reshape(batch, seq, k)
    idxs = jnp.stack([i0o, i1o], axis=2).reshape(batch, seq, k)

    if k < _K:
        pad = _K - k
        scores = jnp.concatenate(
            [scores, jnp.zeros((batch, seq, pad), scores.dtype)], axis=-1)
        idxs = jnp.concatenate(
            [idxs, jnp.zeros((batch, seq, pad), idxs.dtype)], axis=-1)
    half = _K // 2
    return (scores, idxs.astype(jnp.int64), scores[..., :half],
            -scores[..., half:])


# speedup vs baseline: 1.1101x; 1.1101x over previous
import functools

import jax
import jax.numpy as jnp
from jax.experimental import pallas as pl
import jax.experimental.pallas.tpu as pltpu

_K = 32
_NEG_DIAG = -1e9
_NEG_TAKEN = -3e9
_EPR = 4


def _knn_kernel(a_ref, b_ref, scores_ref, idx_ref, s_ref, col_ref,
                *, rblk, seq, k, epr):
    i = pl.program_id(1)
    b_id = pl.program_id(0)

    @pl.when((b_id == 0) & (i == 0))
    def _():
        col_ref[...] = jax.lax.broadcasted_iota(
            jnp.int32, (rblk, seq), 1).astype(jnp.float32)

    a = a_ref[0]
    b = b_ref[0]

    s = jax.lax.dot_general(a, b, (((1,), (1,)), ((), ())),
                            preferred_element_type=jnp.float32)

    col = jax.lax.broadcasted_iota(jnp.int32, (rblk, seq), 1)
    row_g = i * rblk + jax.lax.broadcasted_iota(jnp.int32, (rblk, seq), 0)
    s_ref[...] = jnp.where(col == row_g, _NEG_DIAG, s)

    kcol = jax.lax.broadcasted_iota(jnp.int32, (rblk, k), 1)

    def body(it, carry):
        vals, idxs = carry
        s = s_ref[...]
        colf = col_ref[...]
        for e in range(epr):
            kk = it * epr + e
            m = jnp.max(s, axis=1)
            cand = jnp.where(s >= m[:, None], colf, 3.0e9)
            posf = jnp.min(cand, axis=1)
            s = jnp.where(cand == posf[:, None], _NEG_TAKEN, s)
            pos = posf.astype(jnp.int32)
            sel = kcol == kk
            vals = jnp.where(sel, m[:, None], vals)
            idxs = jnp.where(sel, pos[:, None], idxs)
        s_ref[...] = s
        return vals, idxs

    vals0 = jnp.full((rblk, k), 0.0, jnp.float32)
    idxs0 = jnp.full((rblk, k), 0, jnp.int32)
    vals, idxs = jax.lax.fori_loop(0, k // epr, body, (vals0, idxs0))
    scores_ref[0] = vals
    idx_ref[0] = idxs


@jax.jit
def kernel(embeddings):
    batch, seq, d = embeddings.shape
    k = min(_K, seq - 1)
    rblk = min(512, seq)
    nblk = seq // rblk
    epr = _EPR if k % _EPR == 0 else 1

    emb_n = embeddings / (
        jnp.linalg.norm(embeddings, axis=-1, keepdims=True) + 1e-08)

    kfn = functools.partial(_knn_kernel, rblk=rblk, seq=seq, k=k, epr=epr)
    scores, idxs = pl.pallas_call(
        kfn,
        grid=(batch, nblk),
        in_specs=[
            pl.BlockSpec((1, rblk, d), lambda b, i: (b, i, 0)),
            pl.BlockSpec((1, seq, d), lambda b, i: (b, 0, 0)),
        ],
        out_specs=[
            pl.BlockSpec((1, rblk, k), lambda b, i: (b, i, 0)),
            pl.BlockSpec((1, rblk, k), lambda b, i: (b, i, 0)),
        ],
        out_shape=[
            jax.ShapeDtypeStruct((batch, seq, k), jnp.float32),
            jax.ShapeDtypeStruct((batch, seq, k), jnp.int32),
        ],
        scratch_shapes=[pltpu.VMEM((rblk, seq), jnp.float32),
                        pltpu.VMEM((rblk, seq), jnp.float32)],
    )(emb_n, emb_n)

    if k < _K:
        pad = _K - k
        scores = jnp.concatenate(
            [scores, jnp.zeros((batch, seq, pad), scores.dtype)], axis=-1)
        idxs = jnp.concatenate(
            [idxs, jnp.zeros((batch, seq, pad), idxs.dtype)], axis=-1)
    half = _K // 2
    return (scores, idxs.astype(jnp.int64), scores[..., :half],
            -scores[..., half:])


# speedup vs baseline: 1.1269x; 1.0151x over previous
import functools

import jax
import jax.numpy as jnp
from jax.experimental import pallas as pl
import jax.experimental.pallas.tpu as pltpu

_K = 32
_NEG_DIAG = -1e9
_NEG_TAKEN = -3e9
_EPR = 8


def _knn_kernel(a_ref, b_ref, scores_ref, idx_ref, s_ref, col_ref,
                *, rblk, seq, k, epr):
    i = pl.program_id(1)
    b_id = pl.program_id(0)

    @pl.when((b_id == 0) & (i == 0))
    def _():
        col_ref[...] = jax.lax.broadcasted_iota(
            jnp.int32, (rblk, seq), 1).astype(jnp.float32)

    a = a_ref[0]
    b = b_ref[0]

    s = jax.lax.dot_general(a, b, (((1,), (1,)), ((), ())),
                            preferred_element_type=jnp.float32)

    col = jax.lax.broadcasted_iota(jnp.int32, (rblk, seq), 1)
    row_g = i * rblk + jax.lax.broadcasted_iota(jnp.int32, (rblk, seq), 0)
    s_ref[...] = jnp.where(col == row_g, _NEG_DIAG, s)

    kcol = jax.lax.broadcasted_iota(jnp.int32, (rblk, k), 1)

    def body(it, carry):
        vals, idxs = carry
        s = s_ref[...]
        colf = col_ref[...]
        for e in range(epr):
            kk = it * epr + e
            m = jnp.max(s, axis=1)
            cand = jnp.where(s >= m[:, None], colf, 3.0e9)
            posf = jnp.min(cand, axis=1)
            s = jnp.where(cand == posf[:, None], _NEG_TAKEN, s)
            pos = posf.astype(jnp.int32)
            sel = kcol == kk
            vals = jnp.where(sel, m[:, None], vals)
            idxs = jnp.where(sel, pos[:, None], idxs)
        s_ref[...] = s
        return vals, idxs

    vals0 = jnp.full((rblk, k), 0.0, jnp.float32)
    idxs0 = jnp.full((rblk, k), 0, jnp.int32)
    vals, idxs = jax.lax.fori_loop(0, k // epr, body, (vals0, idxs0))
    scores_ref[0] = vals
    idx_ref[0] = idxs


@jax.jit
def kernel(embeddings):
    batch, seq, d = embeddings.shape
    k = min(_K, seq - 1)
    rblk = min(512, seq)
    nblk = seq // rblk
    epr = _EPR if k % _EPR == 0 else 1

    emb_n = embeddings / (
        jnp.linalg.norm(embeddings, axis=-1, keepdims=True) + 1e-08)

    kfn = functools.partial(_knn_kernel, rblk=rblk, seq=seq, k=k, epr=epr)
    scores, idxs = pl.pallas_call(
        kfn,
        grid=(batch, nblk),
        in_specs=[
            pl.BlockSpec((1, rblk, d), lambda b, i: (b, i, 0)),
            pl.BlockSpec((1, seq, d), lambda b, i: (b, 0, 0)),
        ],
        out_specs=[
            pl.BlockSpec((1, rblk, k), lambda b, i: (b, i, 0)),
            pl.BlockSpec((1, rblk, k), lambda b, i: (b, i, 0)),
        ],
        out_shape=[
            jax.ShapeDtypeStruct((batch, seq, k), jnp.float32),
            jax.ShapeDtypeStruct((batch, seq, k), jnp.int32),
        ],
        scratch_shapes=[pltpu.VMEM((rblk, seq), jnp.float32),
                        pltpu.VMEM((rblk, seq), jnp.float32)],
    )(emb_n, emb_n)

    if k < _K:
        pad = _K - k
        scores = jnp.concatenate(
            [scores, jnp.zeros((batch, seq, pad), scores.dtype)], axis=-1)
        idxs = jnp.concatenate(
            [idxs, jnp.zeros((batch, seq, pad), idxs.dtype)], axis=-1)
    half = _K // 2
    return (scores, idxs.astype(jnp.int64), scores[..., :half],
            -scores[..., half:])
